# baseline (speedup 1.0000x reference)
import functools

import jax
import jax.numpy as jnp
from jax import lax
from jax.experimental import pallas as pl
from jax.experimental.pallas import tpu as pltpu
from jax.experimental.pallas import tpu_sc as plsc

VOCAB_SIZE = 1000000
EMBED_DIM = 32
BATCH = 4096
POS = 200

NTOT = BATCH * POS
CHUNK = 128
NROWS = NTOT // CHUNK
NUM_WORKERS = 32
ROWS_PER_W = NROWS // NUM_WORKERS
GROUP = 10
GROUPS = ROWS_PER_W // GROUP
PAIRS = GROUPS // 2

_mesh = plsc.VectorSubcoreMesh(core_axis_name="c", subcore_axis_name="s")


@functools.partial(
    pl.kernel,
    mesh=_mesh,
    out_type=jax.ShapeDtypeStruct((NROWS, CHUNK, EMBED_DIM), jnp.float32),
    scratch_types=[
        pltpu.VMEM((ROWS_PER_W, CHUNK), jnp.int32),
        pltpu.VMEM((GROUP, CHUNK, EMBED_DIM), jnp.float32),
        pltpu.VMEM((GROUP, CHUNK, EMBED_DIM), jnp.float32),
        pltpu.SemaphoreType.DMA,
        pltpu.SemaphoreType.DMA,
        pltpu.SemaphoreType.DMA,
        pltpu.SemaphoreType.DMA,
    ],
    compiler_params=pltpu.CompilerParams(use_tc_tiling_on_sc=False),
)
def _embed_gather(idx_hbm, table_hbm, out_hbm, idx_v, rows0, rows1,
                  sem_g0, sem_g1, sem_o0, sem_o1):
    wid = lax.axis_index("s") * 2 + lax.axis_index("c")
    rbase = wid * ROWS_PER_W
    pltpu.sync_copy(idx_hbm.at[pl.ds(rbase, ROWS_PER_W)], idx_v)

    def fire_group(g, buf, sem):
        for j in range(GROUP):
            pltpu.async_copy(table_hbm.at[idx_v.at[g * GROUP + j]], buf.at[j], sem)

    def drain(buf, sem):
        pltpu.make_async_copy(out_hbm.at[pl.ds(0, GROUP)], buf, sem).wait()

    fire_group(0, rows0, sem_g0)
    fire_group(1, rows1, sem_g1)

    def body(k, carry):
        g0 = 2 * k
        g1 = 2 * k + 1
        drain(rows0, sem_g0)
        pltpu.async_copy(rows0, out_hbm.at[pl.ds(rbase + g0 * GROUP, GROUP)], sem_o0)
        drain(rows1, sem_g1)
        pltpu.async_copy(rows1, out_hbm.at[pl.ds(rbase + g1 * GROUP, GROUP)], sem_o1)

        @pl.when(k < PAIRS - 1)
        def _():
            drain(rows0, sem_o0)
            fire_group(g0 + 2, rows0, sem_g0)
            drain(rows1, sem_o1)
            fire_group(g1 + 2, rows1, sem_g1)

        return carry

    lax.fori_loop(0, PAIRS, body, 0)
    drain(rows0, sem_o0)
    drain(rows1, sem_o1)


def kernel(x, weight):
    idx = x.reshape(NROWS, CHUNK).astype(jnp.int32)
    out = _embed_gather(idx, weight)
    return out.reshape(BATCH, POS, EMBED_DIM)
